# Initial kernel scaffold; baseline (speedup 1.0000x reference)
import jax
import jax.numpy as jnp
from jax.experimental import pallas as pl


def kernel(x, edge_index, W0, b0, W1, b1):
    raise NotImplementedError("write your pallas kernel here")


# speedup vs baseline: 8.9557x; 8.9557x over previous
import functools

import jax
import jax.numpy as jnp
from jax import lax
from jax.experimental import pallas as pl
from jax.experimental.pallas import tpu as pltpu
from jax.experimental.pallas import tpu_sc as plsc

N = 10000
E = 320000
D = 128

N_PAD = 10240
NC = 2
NSUB = 16
NW = NC * NSUB
CHUNK = 128
CHUNKS = 80
EDGES_PER_W = CHUNKS * CHUNK
E_PAD = NW * EDGES_PER_W
ROWS_PER_T = N_PAD // NSUB

_mesh = plsc.VectorSubcoreMesh(core_axis_name="c", subcore_axis_name="s")


@functools.partial(
    pl.kernel,
    out_type=jax.ShapeDtypeStruct((NW, N_PAD), jnp.float32),
    mesh=_mesh,
    scratch_types=[
        pltpu.VMEM((EDGES_PER_W,), jnp.int32),
        pltpu.VMEM((N_PAD,), jnp.float32),
    ],
    compiler_params=pltpu.CompilerParams(needs_layout_passes=False),
)
def _deg_kernel(dstf_hbm, zeros1_hbm, degp_hbm, dst_v, deg_v):
    c = lax.axis_index("c")
    s = lax.axis_index("s")
    wid = c * NSUB + s
    pltpu.sync_copy(dstf_hbm.at[wid], dst_v)
    pltpu.sync_copy(zeros1_hbm, deg_v)
    ones16 = jnp.ones((16,), jnp.float32)

    def body(j, carry):
        idx = dst_v[pl.ds(j * 16, 16)]
        plsc.addupdate_scatter(deg_v, [idx], ones16)
        return carry

    lax.fori_loop(0, EDGES_PER_W // 16, body, 0)
    pltpu.sync_copy(deg_v, degp_hbm.at[wid])


@functools.partial(
    pl.kernel,
    out_type=jax.ShapeDtypeStruct((NC, N_PAD, D), jnp.float32),
    mesh=_mesh,
    scratch_types=[
        pltpu.VMEM((CHUNKS, CHUNK), jnp.int32),
        pltpu.VMEM((CHUNKS, CHUNK), jnp.int32),
        pltpu.VMEM((CHUNK, D), jnp.float32),
        pltpu.VMEM_SHARED((N_PAD, D), jnp.float32),
        pltpu.SemaphoreType.DMA,
    ],
)
def _scatter_kernel(g_hbm, src_hbm, dst_hbm, zrows_hbm, out_hbm,
                    src_v, dst_v, rows_v, acc_s, sem):
    c = lax.axis_index("c")
    s = lax.axis_index("s")
    wid = c * NSUB + s
    pltpu.sync_copy(src_hbm.at[wid], src_v)
    pltpu.sync_copy(dst_hbm.at[wid], dst_v)
    pltpu.sync_copy(zrows_hbm, acc_s.at[pl.ds(s * ROWS_PER_T, ROWS_PER_T)])
    plsc.subcore_barrier()

    def body(j, carry):
        pltpu.async_copy(g_hbm.at[src_v.at[j]], rows_v, sem).wait()
        pltpu.sync_copy(rows_v, acc_s.at[dst_v.at[j]], add=True)
        return carry

    lax.fori_loop(0, CHUNKS, body, 0)
    plsc.subcore_barrier()
    pltpu.sync_copy(acc_s.at[pl.ds(s * ROWS_PER_T, ROWS_PER_T)],
                    out_hbm.at[c].at[pl.ds(s * ROWS_PER_T, ROWS_PER_T)])


_BLK = 1024
_GRID = N_PAD // _BLK


def _mm0_body(x_ref, w_ref, degp_ref, g_ref, dinv_ref):
    deg = jnp.sum(degp_ref[...], axis=0) + 1.0
    dinv = lax.rsqrt(jnp.maximum(deg, 1.0))
    h = lax.dot_general(x_ref[...], w_ref[...], (((1,), (0,)), ((), ())),
                        preferred_element_type=jnp.float32,
                        precision=lax.Precision.HIGHEST)
    g_ref[...] = h * dinv[:, None]
    dinv_ref[...] = dinv[:, None]


def _mm0(x, W0, degp):
    return pl.pallas_call(
        _mm0_body,
        grid=(_GRID,),
        in_specs=[
            pl.BlockSpec((_BLK, D), lambda i: (i, 0)),
            pl.BlockSpec((D, D), lambda i: (0, 0)),
            pl.BlockSpec((NW, _BLK), lambda i: (0, i)),
        ],
        out_specs=[
            pl.BlockSpec((_BLK, D), lambda i: (i, 0)),
            pl.BlockSpec((_BLK, 1), lambda i: (i, 0)),
        ],
        out_shape=[
            jax.ShapeDtypeStruct((N_PAD, D), jnp.float32),
            jax.ShapeDtypeStruct((N_PAD, 1), jnp.float32),
        ],
    )(x, W0, degp)


def _mid_body(p0_ref, p1_ref, g0_ref, dinv_ref, b0_ref, w1_ref, g1_ref):
    dinv = dinv_ref[...]
    acc = p0_ref[...] + p1_ref[...] + g0_ref[...]
    h = jnp.maximum(acc * dinv + b0_ref[...], 0.0)
    h1 = lax.dot_general(h, w1_ref[...], (((1,), (0,)), ((), ())),
                         preferred_element_type=jnp.float32,
                         precision=lax.Precision.HIGHEST)
    g1_ref[...] = h1 * dinv


def _mid(p0, p1, g0, dinv, b0, W1):
    return pl.pallas_call(
        _mid_body,
        grid=(_GRID,),
        in_specs=[
            pl.BlockSpec((_BLK, D), lambda i: (i, 0)),
            pl.BlockSpec((_BLK, D), lambda i: (i, 0)),
            pl.BlockSpec((_BLK, D), lambda i: (i, 0)),
            pl.BlockSpec((_BLK, 1), lambda i: (i, 0)),
            pl.BlockSpec((1, D), lambda i: (0, 0)),
            pl.BlockSpec((D, D), lambda i: (0, 0)),
        ],
        out_specs=pl.BlockSpec((_BLK, D), lambda i: (i, 0)),
        out_shape=jax.ShapeDtypeStruct((N_PAD, D), jnp.float32),
    )(p0, p1, g0, dinv, b0, W1)


def _final_body(q0_ref, q1_ref, g1_ref, dinv_ref, b1_ref, out_ref):
    acc = q0_ref[...] + q1_ref[...] + g1_ref[...]
    out_ref[...] = acc * dinv_ref[...] + b1_ref[...]


def _final(q0, q1, g1, dinv, b1):
    return pl.pallas_call(
        _final_body,
        grid=(_GRID,),
        in_specs=[
            pl.BlockSpec((_BLK, D), lambda i: (i, 0)),
            pl.BlockSpec((_BLK, D), lambda i: (i, 0)),
            pl.BlockSpec((_BLK, D), lambda i: (i, 0)),
            pl.BlockSpec((_BLK, 1), lambda i: (i, 0)),
            pl.BlockSpec((1, D), lambda i: (0, 0)),
        ],
        out_specs=pl.BlockSpec((_BLK, D), lambda i: (i, 0)),
        out_shape=jax.ShapeDtypeStruct((N_PAD, D), jnp.float32),
    )(q0, q1, g1, dinv, b1)


@jax.jit
def kernel(x, edge_index, W0, b0, W1, b1):
    x_pad = jnp.concatenate(
        [x, jnp.zeros((N_PAD - N, D), jnp.float32)], axis=0)
    pad = jnp.full((E_PAD - E,), N, jnp.int32)
    src = jnp.concatenate([edge_index[0], pad]).reshape(NW, CHUNKS, CHUNK)
    dst_flat = jnp.concatenate([edge_index[1], pad])
    dst = dst_flat.reshape(NW, CHUNKS, CHUNK)
    dstf = dst_flat.reshape(NW, EDGES_PER_W)
    zeros1 = jnp.zeros((N_PAD,), jnp.float32)
    zrows = jnp.zeros((ROWS_PER_T, D), jnp.float32)
    b0r = b0.reshape(1, D)
    b1r = b1.reshape(1, D)

    degp = _deg_kernel(dstf, zeros1)
    g0, dinv = _mm0(x_pad, W0, degp)
    p = _scatter_kernel(g0, src, dst, zrows)
    g1 = _mid(p[0], p[1], g0, dinv, b0r, W1)
    q = _scatter_kernel(g1, src, dst, zrows)
    out = _final(q[0], q[1], g1, dinv, b1r)
    return out[:N]
